# Initial kernel scaffold; baseline (speedup 1.0000x reference)
import jax
import jax.numpy as jnp
from jax.experimental import pallas as pl


def kernel(x, edge_index, edge_weight, W, b):
    raise NotImplementedError("write your pallas kernel here")


# speedup vs baseline: 3.9933x; 3.9933x over previous
import functools

import jax
import jax.numpy as jnp
from jax import lax
from jax.experimental import pallas as pl
from jax.experimental.pallas import tpu as pltpu
from jax.experimental.pallas import tpu_sc as plsc

NC = 2
NS = 16
NW = NC * NS
C = 128
L = 16


def _sc_aggregate(h, src3, dst3, ew2, n_nodes, d, ch):
    zrows = 80
    nzch = n_nodes // zrows
    nzt = (nzch + NS - 1) // NS
    mesh = plsc.VectorSubcoreMesh(core_axis_name="c", subcore_axis_name="s")

    @functools.partial(
        pl.kernel,
        out_type=jax.ShapeDtypeStruct((NC, n_nodes, d), jnp.float32),
        mesh=mesh,
        scratch_types=[
            pltpu.VMEM((ch, C), jnp.int32),
            pltpu.VMEM((ch, C), jnp.int32),
            pltpu.VMEM((ch * C,), jnp.float32),
            pltpu.VMEM((C, d), jnp.float32),
            pltpu.VMEM_SHARED((n_nodes, d), jnp.float32),
            pltpu.SemaphoreType.DMA,
        ],
    )
    def k(h_hbm, src_hbm, dst_hbm, ew_hbm, out_hbm,
          src_v, dst_v, ew_v, rows_v, acc, sem):
        c = lax.axis_index("c")
        s = lax.axis_index("s")
        w = s * NC + c

        pltpu.sync_copy(src_hbm.at[w], src_v)
        pltpu.sync_copy(dst_hbm.at[w], dst_v)
        pltpu.sync_copy(ew_hbm.at[w], ew_v)

        z16 = jnp.zeros((L,), jnp.float32)

        def zero_row(i, carry):
            for kk in range(d // L):
                rows_v[i, pl.ds(L * kk, L)] = z16
            return carry

        lax.fori_loop(0, zrows, zero_row, 0)

        def zero_chunk(t, carry):
            idx = s + t * NS

            @pl.when(idx < nzch)
            def _():
                pltpu.sync_copy(rows_v.at[pl.ds(0, zrows)],
                                acc.at[pl.ds(idx * zrows, zrows)])

            return carry

        lax.fori_loop(0, nzt, zero_chunk, 0)
        plsc.subcore_barrier()

        def chunk_body(ci, carry):
            pltpu.async_copy(h_hbm.at[src_v.at[ci]], rows_v, sem).wait()

            def group_body(g, c2):
                wv = ew_v[pl.ds(ci * C + g * L, L)]
                for j in range(L):
                    w16 = jnp.zeros((L,), jnp.float32) + wv[j]
                    e = g * L + j
                    for kk in range(d // L):
                        sl = pl.ds(L * kk, L)
                        rows_v[e, sl] = rows_v[e, sl] * w16
                return c2

            lax.fori_loop(0, C // L, group_body, 0)
            pltpu.sync_copy(rows_v, acc.at[dst_v.at[ci]], add=True)
            return carry

        lax.fori_loop(0, ch, chunk_body, 0)
        plsc.subcore_barrier()

        def write_chunk(t, carry):
            idx = s + t * NS

            @pl.when(idx < nzch)
            def _():
                pltpu.sync_copy(acc.at[pl.ds(idx * zrows, zrows)],
                                out_hbm.at[c, pl.ds(idx * zrows, zrows)])

            return carry

        lax.fori_loop(0, nzt, write_chunk, 0)

    return k(h, src3, dst3, ew2)


def _mm_first(x, W, b2):
    def body(x_ref, w_ref, b_ref, o_ref):
        o_ref[...] = jnp.dot(x_ref[...], w_ref[...],
                             preferred_element_type=jnp.float32) + b_ref[...]

    return pl.pallas_call(
        body, out_shape=jax.ShapeDtypeStruct(x.shape, jnp.float32),
    )(x, W, b2)


def _mm_fused(agg, W, b2):
    n, d = agg.shape[1], agg.shape[2]

    def body(a_ref, w_ref, b_ref, o_ref):
        xr = jnp.maximum(a_ref[0] + a_ref[1], 0.0)
        o_ref[...] = jnp.dot(xr, w_ref[...],
                             preferred_element_type=jnp.float32) + b_ref[...]

    return pl.pallas_call(
        body, out_shape=jax.ShapeDtypeStruct((n, d), jnp.float32),
    )(agg, W, b2)


def _combine(agg):
    n, d = agg.shape[1], agg.shape[2]

    def body(a_ref, o_ref):
        o_ref[...] = jnp.maximum(a_ref[0] + a_ref[1], 0.0)

    return pl.pallas_call(
        body, out_shape=jax.ShapeDtypeStruct((n, d), jnp.float32),
    )(agg)


def kernel(x, edge_index, edge_weight, W, b):
    n, d = x.shape
    e = edge_weight.shape[0]
    epad = ((e + NW * C - 1) // (NW * C)) * (NW * C)
    ch = epad // (NW * C)
    pad = epad - e
    src3 = jnp.pad(edge_index[0], (0, pad)).reshape(NW, ch, C)
    dst3 = jnp.pad(edge_index[1], (0, pad)).reshape(NW, ch, C)
    ew2 = jnp.pad(edge_weight, (0, pad)).reshape(NW, ch * C)
    b2 = b.reshape(1, d)

    h = _mm_first(x, W, b2)
    agg = _sc_aggregate(h, src3, dst3, ew2, n, d, ch)
    for _ in range(2):
        h = _mm_fused(agg, W, b2)
        agg = _sc_aggregate(h, src3, dst3, ew2, n, d, ch)
    return _combine(agg)
